# baseline (speedup 1.0000x reference)
import functools

import jax
import jax.numpy as jnp
from jax import lax
from jax.experimental import pallas as pl
from jax.experimental.pallas import tpu as pltpu
from jax.experimental.pallas import tpu_sc as plsc

N_NODES = 50000
N_EDGES = 800000
H = 64

NP = 50176
NC, NS, L = 2, 16, 16
NW = NC * NS
EROW = 128
E_ROWS = 6400
E_PAD = E_ROWS * EROW
KG = 8
KG1 = 8
R = 3136
GR = NP // R

_PREC = lax.Precision.HIGHEST
NPT = NP // NW
NSC = NPT // 2


def _mesh():
    return plsc.VectorSubcoreMesh(core_axis_name="c", subcore_axis_name="s")


def _deg_body(dst2_hbm, out_hbm, hist, idxb):
    c = lax.axis_index("c")
    s = lax.axis_index("s")
    wid = s * NC + c
    zv = jnp.zeros((L,), jnp.float32)

    def _z(i, carry):
        hist[pl.ds(i * L, L)] = zv
        return carry

    lax.fori_loop(0, NP // L, _z, 0)

    ones = jnp.ones((L,), jnp.float32)
    rows_per = E_ROWS // NW

    def _outer(o, carry):
        rb = wid * rows_per + o * KG
        pltpu.sync_copy(dst2_hbm.at[pl.ds(rb, KG)], idxb)
        for k in range(KG):
            for g in range(EROW // L):
                idx = idxb[k, pl.ds(g * L, L)]
                plsc.addupdate_scatter(hist, [idx], ones)
        return carry

    lax.fori_loop(0, rows_per // KG, _outer, 0)
    pltpu.sync_copy(hist, out_hbm.at[wid, 0])


_deg = functools.partial(
    pl.kernel,
    out_type=jax.ShapeDtypeStruct((NW, 1, NP), jnp.float32),
    mesh=_mesh(),
    compiler_params=pltpu.CompilerParams(needs_layout_passes=False, use_tc_tiling_on_sc=False),
    scratch_types=[
        pltpu.VMEM((NP,), jnp.float32),
        pltpu.VMEM((KG, EROW), jnp.int32),
    ],
)(_deg_body)


def _vrsqrt(d):
    magic = jnp.full((L,), 0x5F3759DF, jnp.int32)
    y = plsc.bitcast(magic - (plsc.bitcast(d, jnp.int32) >> 1), jnp.float32)
    for _ in range(4):
        y = y * (1.5 - 0.5 * d * y * y)
    return y


def _norm_body(degp_hbm, xpf_hbm, dinv_hbm, uf_hbm, dacc, dtmp, dinvb, ubf):
    c = lax.axis_index("c")
    s = lax.axis_index("s")
    wid = s * NC + c
    nbase = wid * NPT
    pltpu.sync_copy(degp_hbm.at[0, 0, pl.ds(nbase, NPT)], dacc)

    def _w(w, carry):
        pltpu.sync_copy(degp_hbm.at[w, 0, pl.ds(nbase, NPT)], dtmp)

        def _g(g, inner):
            dacc[pl.ds(g * L, L)] = dacc[pl.ds(g * L, L)] + dtmp[pl.ds(g * L, L)]
            return inner

        lax.fori_loop(0, NPT // L, _g, 0)
        return carry

    lax.fori_loop(1, NW, _w, 0)

    def _r(g, carry):
        d = dacc[pl.ds(g * L, L)] + 1.0
        dinvb[pl.ds(g * L, L)] = _vrsqrt(d)
        return carry

    lax.fori_loop(0, NPT // L, _r, 0)

    pltpu.sync_copy(xpf_hbm.at[pl.ds(8 * nbase, 8 * NPT)], ubf)
    lane = lax.iota(jnp.int32, L)
    low8 = lane < 8

    def _u(g, carry):
        v = dinvb[pl.ds(g * L, L)]
        for j in range(8):
            dp = jnp.where(low8, jnp.full((L,), v[2 * j]),
                           jnp.full((L,), v[2 * j + 1]))
            off = g * 8 * L + j * L
            ubf[pl.ds(off, L)] = ubf[pl.ds(off, L)] * dp
        return carry

    lax.fori_loop(0, NPT // L, _u, 0)
    pltpu.sync_copy(dinvb, dinv_hbm.at[pl.ds(nbase, NPT)])
    pltpu.sync_copy(ubf, uf_hbm.at[pl.ds(8 * nbase, 8 * NPT)])


_norm = functools.partial(
    pl.kernel,
    out_type=(jax.ShapeDtypeStruct((NP,), jnp.float32),
              jax.ShapeDtypeStruct((NP * 8,), jnp.float32)),
    mesh=_mesh(),
    compiler_params=pltpu.CompilerParams(needs_layout_passes=False, use_tc_tiling_on_sc=False),
    scratch_types=[
        pltpu.VMEM((NPT,), jnp.float32),
        pltpu.VMEM((NPT,), jnp.float32),
        pltpu.VMEM((NPT,), jnp.float32),
        pltpu.VMEM((NPT * 8,), jnp.float32),
    ],
)(_norm_body)


def _pipe_gather_scatter(tbl_hbm, si_hbm, di_hbm, base, ng, kg,
                         idxs, idxd, rows, acc, sems):
    def _load(g, b):
        pltpu.sync_copy(si_hbm.at[pl.ds(base + g * kg, kg)], idxs.at[b])
        pltpu.sync_copy(di_hbm.at[pl.ds(base + g * kg, kg)], idxd.at[b])

    def _descs(b):
        return [
            pltpu.make_async_copy(tbl_hbm.at[idxs.at[b, k]], rows.at[b, k],
                                  sems[b])
            for k in range(kg)
        ]

    def _scat(b):
        for k in range(kg):
            pltpu.sync_copy(rows.at[b, k], acc.at[idxd.at[b, k]], add=True)

    _load(0, 0)
    for d in _descs(0):
        d.start()
    _load(1, 1)
    n2 = ng // 2

    def _pair(i, carry):
        d1 = _descs(1)
        for d in d1:
            d.start()
        d0 = _descs(0)
        for d in d0:
            d.wait()
        _scat(0)

        @pl.when(2 * i + 2 < ng)
        def _():
            _load(2 * i + 2, 0)
            for d in _descs(0):
                d.start()

        for d in d1:
            d.wait()
        _scat(1)

        @pl.when(2 * i + 3 < ng)
        def _():
            _load(2 * i + 3, 1)

        return carry

    lax.fori_loop(0, n2, _pair, 0)
    if ng % 2:
        for d in _descs(0):
            d.wait()
        _scat(0)


def _agg1_body(src2_hbm, dst2_hbm, u_hbm, z8_hbm, p1_hbm,
               idxs, idxd, rows, acc, sem0, sem1):
    c = lax.axis_index("c")
    s = lax.axis_index("s")
    wid = s * NC + c

    @pl.when(jnp.logical_and(s == 0, c == 0))
    def _():
        pltpu.sync_copy(u_hbm, acc)

    @pl.when(jnp.logical_and(s == 0, c != 0))
    def _():
        pltpu.sync_copy(z8_hbm, acc)

    plsc.subcore_barrier()
    rows_per = E_ROWS // NW
    _pipe_gather_scatter(u_hbm, src2_hbm, dst2_hbm, wid * rows_per,
                         rows_per // KG1, KG1, idxs, idxd, rows, acc,
                         (sem0, sem1))
    plsc.subcore_barrier()

    @pl.when(s == 0)
    def _():
        pltpu.sync_copy(acc, p1_hbm.at[c])


_agg1 = functools.partial(
    pl.kernel,
    out_type=jax.ShapeDtypeStruct((NC, NP, 8), jnp.float32),
    mesh=_mesh(),
    compiler_params=pltpu.CompilerParams(needs_layout_passes=False, use_tc_tiling_on_sc=False),
    scratch_types=[
        pltpu.VMEM((2, KG1, EROW), jnp.int32),
        pltpu.VMEM((2, KG1, EROW), jnp.int32),
        pltpu.VMEM((2, KG1, EROW, 8), jnp.float32),
        pltpu.VMEM_SHARED((NP, 8), jnp.float32),
        pltpu.SemaphoreType.DMA,
        pltpu.SemaphoreType.DMA,
    ],
)(_agg1_body)


def _dense1_body(p1f_hbm, dinv_hbm, w1t_hbm, b1_hbm, g1_hbm,
                 s8b, tmp8, dinvb, gbuf, wb, bb):
    c = lax.axis_index("c")
    s = lax.axis_index("s")
    wid = s * NC + c
    pltpu.sync_copy(w1t_hbm, wb)
    pltpu.sync_copy(b1_hbm, bb)
    wv = [[wb[k, pl.ds(16 * q, L)] for q in range(4)] for k in range(3)]
    bv = [bb[pl.ds(16 * q, L)] for q in range(4)]

    for half in range(2):
        nbase = wid * NPT + half * NSC
        pltpu.sync_copy(p1f_hbm.at[0, pl.ds(8 * nbase, 8 * NSC)], s8b)
        pltpu.sync_copy(p1f_hbm.at[1, pl.ds(8 * nbase, 8 * NSC)], tmp8)
        pltpu.sync_copy(dinv_hbm.at[pl.ds(nbase, NSC)], dinvb)

        def _add(g, carry):
            s8b[pl.ds(g * L, L)] = s8b[pl.ds(g * L, L)] + tmp8[pl.ds(g * L, L)]
            return carry

        lax.fori_loop(0, NSC * 8 // L, _add, 0)

        def _node16(i, carry):
            dv16 = dinvb[pl.ds(i * L, L)]
            rowv = [s8b[pl.ds(i * 8 * L + m * L, L)] for m in range(8)]
            for j in range(L):
                r = rowv[j // 2]
                o = (j % 2) * 8
                dv = dv16[j]
                t0 = jnp.full((L,), dv * r[o])
                t1 = jnp.full((L,), dv * r[o + 1])
                t2 = jnp.full((L,), dv * r[o + 2])
                dvb = jnp.full((L,), dv)
                for q in range(4):
                    h = t0 * wv[0][q] + t1 * wv[1][q] + t2 * wv[2][q] + bv[q]
                    gbuf[q, pl.ds((i * L + j) * L, L)] = dvb * jnp.maximum(h, 0.0)
            return carry

        lax.fori_loop(0, NSC // L, _node16, 0)
        for q in range(4):
            pltpu.sync_copy(
                gbuf.at[q],
                g1_hbm.at[pl.ds(16 * (q * NP + nbase), 16 * NSC)])


_dense1 = functools.partial(
    pl.kernel,
    out_type=jax.ShapeDtypeStruct((4 * NP * 16,), jnp.float32),
    mesh=_mesh(),
    compiler_params=pltpu.CompilerParams(needs_layout_passes=False, use_tc_tiling_on_sc=False),
    scratch_types=[
        pltpu.VMEM((NSC * 8,), jnp.float32),
        pltpu.VMEM((NSC * 8,), jnp.float32),
        pltpu.VMEM((NSC,), jnp.float32),
        pltpu.VMEM((4, NSC * 16), jnp.float32),
        pltpu.VMEM((3, H), jnp.float32),
        pltpu.VMEM((H,), jnp.float32),
    ],
)(_dense1_body)


def _agg2_body(src4_hbm, dst2_hbm, g1f_hbm, agg2_hbm,
               idxs, idxd, rows, acc, sem0, sem1):
    c = lax.axis_index("c")
    s = lax.axis_index("s")
    rows_per = E_ROWS // NS

    for p in range(2):
        @pl.when(s == 0)
        def _(p=p):
            q = 2 * p + c
            pltpu.sync_copy(g1f_hbm.at[pl.ds(q * NP, NP)], acc)

        plsc.subcore_barrier()
        _pipe_gather_scatter(g1f_hbm, src4_hbm.at[2 * p + c], dst2_hbm,
                             s * rows_per, rows_per // KG, KG,
                             idxs, idxd, rows, acc, (sem0, sem1))
        plsc.subcore_barrier()

        @pl.when(s == 0)
        def _():
            pltpu.sync_copy(acc, agg2_hbm.at[p, c])

        plsc.subcore_barrier()


_agg2 = functools.partial(
    pl.kernel,
    out_type=jax.ShapeDtypeStruct((2, NC, NP, 16), jnp.float32),
    mesh=_mesh(),
    compiler_params=pltpu.CompilerParams(needs_layout_passes=False, use_tc_tiling_on_sc=False),
    scratch_types=[
        pltpu.VMEM((2, KG, EROW), jnp.int32),
        pltpu.VMEM((2, KG, EROW), jnp.int32),
        pltpu.VMEM((2, KG, EROW, 16), jnp.float32),
        pltpu.VMEM_SHARED((NP, 16), jnp.float32),
        pltpu.SemaphoreType.DMA,
        pltpu.SemaphoreType.DMA,
    ],
)(_agg2_body)


def _pq_body(agg4_hbm, dinv_hbm, m2_hbm, p_hbm, q_hbm,
             st, dinvb, pbuf, qbuf, m2b):
    c = lax.axis_index("c")
    s = lax.axis_index("s")
    wid = s * NC + c
    nbase = wid * NPT
    pltpu.sync_copy(m2_hbm, m2b)
    m2p = [m2b[0, pl.ds(16 * qi, L)] for qi in range(4)]
    m2q = [m2b[1, pl.ds(16 * qi, L)] for qi in range(4)]
    for qi in range(4):
        pltpu.sync_copy(agg4_hbm.at[qi, pl.ds(16 * nbase, 16 * NPT)],
                        st.at[qi])
    pltpu.sync_copy(dinv_hbm.at[pl.ds(nbase, NPT)], dinvb)
    lane = lax.iota(jnp.int32, L)

    def _blk(i, carry):
        dv16 = dinvb[pl.ds(i * L, L)]
        pvec = jnp.zeros((L,), jnp.float32)
        qvec = jnp.zeros((L,), jnp.float32)
        for j in range(L):
            dvb = jnp.full((L,), dv16[j])
            off = (i * L + j) * L
            accp = accq = None
            for qi in range(4):
                pre = dvb * st[qi, pl.ds(off, L)]
                tp = pre * m2p[qi]
                tq = pre * m2q[qi]
                accp = tp if accp is None else accp + tp
                accq = tq if accq is None else accq + tq
            pn = jnp.sum(accp)
            qn = jnp.sum(accq)
            m = lane == j
            pvec = jnp.where(m, jnp.full((L,), pn), pvec)
            qvec = jnp.where(m, jnp.full((L,), qn), qvec)
        pbuf[pl.ds(i * L, L)] = pvec
        qbuf[pl.ds(i * L, L)] = qvec
        return carry

    lax.fori_loop(0, NPT // L, _blk, 0)
    pltpu.sync_copy(pbuf, p_hbm.at[pl.ds(nbase, NPT)])
    pltpu.sync_copy(qbuf, q_hbm.at[pl.ds(nbase, NPT)])


_pq = functools.partial(
    pl.kernel,
    out_type=(jax.ShapeDtypeStruct((NP,), jnp.float32),
              jax.ShapeDtypeStruct((NP,), jnp.float32)),
    mesh=_mesh(),
    compiler_params=pltpu.CompilerParams(needs_layout_passes=False, use_tc_tiling_on_sc=False),
    scratch_types=[
        pltpu.VMEM((4, NPT * 16), jnp.float32),
        pltpu.VMEM((NPT,), jnp.float32),
        pltpu.VMEM((NPT,), jnp.float32),
        pltpu.VMEM((NPT,), jnp.float32),
        pltpu.VMEM((2, H), jnp.float32),
    ],
)(_pq_body)


CH = 2000
NCHUNK = N_EDGES // CH


def _edge_body(src_hbm, dst_hbm, ea0_hbm, ea1_hbm, p_hbm, q_hbm, cst_hbm,
               out_hbm, p_v, q_v, sbuf, dbuf, e0, e1, ob, cst_v):
    c = lax.axis_index("c")
    s = lax.axis_index("s")
    wid = s * NC + c
    pltpu.sync_copy(p_hbm, p_v)
    pltpu.sync_copy(q_hbm, q_v)
    pltpu.sync_copy(cst_hbm, cst_v)
    w0 = cst_v[0, :]
    w1 = cst_v[1, :]
    bcv = cst_v[2, :]

    def _chunk(j, carry):
        base = (wid + j * NW) * CH
        pltpu.sync_copy(src_hbm.at[pl.ds(base, CH)], sbuf)
        pltpu.sync_copy(dst_hbm.at[pl.ds(base, CH)], dbuf)
        pltpu.sync_copy(ea0_hbm.at[pl.ds(base, CH)], e0)
        pltpu.sync_copy(ea1_hbm.at[pl.ds(base, CH)], e1)

        def _grp(g, inner):
            off = g * L
            si = sbuf[pl.ds(off, L)]
            di = dbuf[pl.ds(off, L)]
            pv = plsc.load_gather(p_v, [si])
            qv = plsc.load_gather(q_v, [di])
            z = pv + qv + w0 * e0[pl.ds(off, L)] + w1 * e1[pl.ds(off, L)] + bcv
            ob[pl.ds(off, L)] = 1.0 / (1.0 + jnp.exp(-z))
            return inner

        lax.fori_loop(0, CH // L, _grp, 0)
        pltpu.sync_copy(ob, out_hbm.at[pl.ds(base, CH)])
        return carry

    nmine = (NCHUNK - 1 - wid) // NW + 1
    lax.fori_loop(0, nmine, _chunk, 0)


_edges = functools.partial(
    pl.kernel,
    out_type=jax.ShapeDtypeStruct((N_EDGES,), jnp.float32),
    mesh=_mesh(),
    compiler_params=pltpu.CompilerParams(needs_layout_passes=False, use_tc_tiling_on_sc=False),
    scratch_types=[
        pltpu.VMEM((NP,), jnp.float32),
        pltpu.VMEM((NP,), jnp.float32),
        pltpu.VMEM((CH,), jnp.int32),
        pltpu.VMEM((CH,), jnp.int32),
        pltpu.VMEM((CH,), jnp.float32),
        pltpu.VMEM((CH,), jnp.float32),
        pltpu.VMEM((CH,), jnp.float32),
        pltpu.VMEM((3, L), jnp.float32),
    ],
)(_edge_body)


def kernel(x, edge_index, edge_attr, W1, b1, W2, b2, Wc, bc):
    src = edge_index[0]
    dst = edge_index[1]
    padi = jnp.full((E_PAD - N_EDGES,), NP - 1, jnp.int32)
    src2 = jnp.concatenate([src, padi]).reshape(E_ROWS, EROW)
    dst2 = jnp.concatenate([dst, padi]).reshape(E_ROWS, EROW)
    src4 = jnp.stack([src2, src2 + NP, src2 + 2 * NP, src2 + 3 * NP])
    xpf = jnp.zeros((NP, 8), jnp.float32).at[:N_NODES, :3].set(x).reshape(NP * 8)
    z8 = jnp.zeros((NP, 8), jnp.float32)

    degp = _deg(dst2)
    dinvv, uf = _norm(degp, xpf)
    u2d = uf.reshape(NP, 8)
    p1 = _agg1(src2, dst2, u2d, z8)
    p1f = p1.reshape(NC, NP * 8)
    g1fl = _dense1(p1f, dinvv, W1.T, b1)
    g1f = g1fl.reshape(4 * NP, 16)
    agg4 = _agg2(src4, dst2, g1f).reshape(4, NP * 16)
    wsd = Wc[0, :2 * H].reshape(2, H)
    m2 = jnp.dot(wsd, W2, precision=_PREC)
    pvec, qvec = _pq(agg4, dinvv, m2)
    bconst = bc[0] + jnp.dot(b2, wsd[0]) + jnp.dot(b2, wsd[1])
    cst = jnp.stack([
        jnp.full((L,), Wc[0, 2 * H], jnp.float32),
        jnp.full((L,), Wc[0, 2 * H + 1], jnp.float32),
        jnp.full((L,), bconst, jnp.float32),
    ])
    out = _edges(src, dst, edge_attr[:, 0], edge_attr[:, 1],
                 pvec, qvec, cst)
    return out[:, None]


# speedup vs baseline: 1.0049x; 1.0049x over previous
import functools

import jax
import jax.numpy as jnp
from jax import lax
from jax.experimental import pallas as pl
from jax.experimental.pallas import tpu as pltpu
from jax.experimental.pallas import tpu_sc as plsc

N_NODES = 50000
N_EDGES = 800000
H = 64

NP = 50176
NC, NS, L = 2, 16, 16
NW = NC * NS
EROW = 128
E_ROWS = 6400
E_PAD = E_ROWS * EROW
KG = 8
KG1 = 8
R = 3136
GR = NP // R

_PREC = lax.Precision.HIGHEST
NPT = NP // NW
NSC = NPT // 2


def _mesh():
    return plsc.VectorSubcoreMesh(core_axis_name="c", subcore_axis_name="s")


def _deg_body(dst2_hbm, out_hbm, hist, idxb):
    c = lax.axis_index("c")
    s = lax.axis_index("s")
    wid = s * NC + c
    zv = jnp.zeros((L,), jnp.float32)

    def _z(i, carry):
        hist[pl.ds(i * L, L)] = zv
        return carry

    lax.fori_loop(0, NP // L, _z, 0)

    ones = jnp.ones((L,), jnp.float32)
    rows_per = E_ROWS // NW

    def _outer(o, carry):
        rb = wid * rows_per + o * KG
        pltpu.sync_copy(dst2_hbm.at[pl.ds(rb, KG)], idxb)
        for k in range(KG):
            for g in range(EROW // L):
                idx = idxb[k, pl.ds(g * L, L)]
                plsc.addupdate_scatter(hist, [idx], ones)
        return carry

    lax.fori_loop(0, rows_per // KG, _outer, 0)
    pltpu.sync_copy(hist, out_hbm.at[wid, 0])


_deg = functools.partial(
    pl.kernel,
    out_type=jax.ShapeDtypeStruct((NW, 1, NP), jnp.float32),
    mesh=_mesh(),
    compiler_params=pltpu.CompilerParams(needs_layout_passes=False, use_tc_tiling_on_sc=False),
    scratch_types=[
        pltpu.VMEM((NP,), jnp.float32),
        pltpu.VMEM((KG, EROW), jnp.int32),
    ],
)(_deg_body)


def _vrsqrt(d):
    magic = jnp.full((L,), 0x5F3759DF, jnp.int32)
    y = plsc.bitcast(magic - (plsc.bitcast(d, jnp.int32) >> 1), jnp.float32)
    for _ in range(4):
        y = y * (1.5 - 0.5 * d * y * y)
    return y


def _norm_body(degp_hbm, x0_hbm, x1_hbm, x2_hbm, dinv_hbm, uf_hbm,
               dacc, dtmp, dinvb, ubf, c0b, c1b, c2b):
    c = lax.axis_index("c")
    s = lax.axis_index("s")
    wid = s * NC + c
    nbase = wid * NPT
    pltpu.sync_copy(degp_hbm.at[0, 0, pl.ds(nbase, NPT)], dacc)

    def _w(w, carry):
        pltpu.sync_copy(degp_hbm.at[w, 0, pl.ds(nbase, NPT)], dtmp)

        def _g(g, inner):
            dacc[pl.ds(g * L, L)] = dacc[pl.ds(g * L, L)] + dtmp[pl.ds(g * L, L)]
            return inner

        lax.fori_loop(0, NPT // L, _g, 0)
        return carry

    lax.fori_loop(1, NW, _w, 0)

    def _r(g, carry):
        d = dacc[pl.ds(g * L, L)] + 1.0
        dinvb[pl.ds(g * L, L)] = _vrsqrt(d)
        return carry

    lax.fori_loop(0, NPT // L, _r, 0)

    pltpu.sync_copy(x0_hbm.at[pl.ds(nbase, NPT)], c0b)
    pltpu.sync_copy(x1_hbm.at[pl.ds(nbase, NPT)], c1b)
    pltpu.sync_copy(x2_hbm.at[pl.ds(nbase, NPT)], c2b)
    lane = lax.iota(jnp.int32, L)

    def _u(g, carry):
        v = dinvb[pl.ds(g * L, L)]
        x0 = c0b[pl.ds(g * L, L)]
        x1 = c1b[pl.ds(g * L, L)]
        x2 = c2b[pl.ds(g * L, L)]
        for j in range(8):
            na, nb = 2 * j, 2 * j + 1
            z = jnp.zeros((L,), jnp.float32)
            z = jnp.where(lane == 0, jnp.full((L,), v[na] * x0[na]), z)
            z = jnp.where(lane == 1, jnp.full((L,), v[na] * x1[na]), z)
            z = jnp.where(lane == 2, jnp.full((L,), v[na] * x2[na]), z)
            z = jnp.where(lane == 8, jnp.full((L,), v[nb] * x0[nb]), z)
            z = jnp.where(lane == 9, jnp.full((L,), v[nb] * x1[nb]), z)
            z = jnp.where(lane == 10, jnp.full((L,), v[nb] * x2[nb]), z)
            ubf[pl.ds(g * 8 * L + j * L, L)] = z
        return carry

    lax.fori_loop(0, NPT // L, _u, 0)
    pltpu.sync_copy(dinvb, dinv_hbm.at[pl.ds(nbase, NPT)])
    pltpu.sync_copy(ubf, uf_hbm.at[pl.ds(8 * nbase, 8 * NPT)])


_norm = functools.partial(
    pl.kernel,
    out_type=(jax.ShapeDtypeStruct((NP,), jnp.float32),
              jax.ShapeDtypeStruct((NP * 8,), jnp.float32)),
    mesh=_mesh(),
    compiler_params=pltpu.CompilerParams(needs_layout_passes=False, use_tc_tiling_on_sc=False),
    scratch_types=[
        pltpu.VMEM((NPT,), jnp.float32),
        pltpu.VMEM((NPT,), jnp.float32),
        pltpu.VMEM((NPT,), jnp.float32),
        pltpu.VMEM((NPT * 8,), jnp.float32),
        pltpu.VMEM((NPT,), jnp.float32),
        pltpu.VMEM((NPT,), jnp.float32),
        pltpu.VMEM((NPT,), jnp.float32),
    ],
)(_norm_body)


def _pipe_gather_scatter(tbl_hbm, si_hbm, di_hbm, base, ng, kg,
                         idxs, idxd, rows, acc, sems):
    def _load(g, b):
        pltpu.sync_copy(si_hbm.at[pl.ds(base + g * kg, kg)], idxs.at[b])
        pltpu.sync_copy(di_hbm.at[pl.ds(base + g * kg, kg)], idxd.at[b])

    def _descs(b):
        return [
            pltpu.make_async_copy(tbl_hbm.at[idxs.at[b, k]], rows.at[b, k],
                                  sems[b])
            for k in range(kg)
        ]

    def _scat(b):
        for k in range(kg):
            pltpu.sync_copy(rows.at[b, k], acc.at[idxd.at[b, k]], add=True)

    _load(0, 0)
    for d in _descs(0):
        d.start()
    _load(1, 1)
    n2 = ng // 2

    def _pair(i, carry):
        d1 = _descs(1)
        for d in d1:
            d.start()
        d0 = _descs(0)
        for d in d0:
            d.wait()
        _scat(0)

        @pl.when(2 * i + 2 < ng)
        def _():
            _load(2 * i + 2, 0)
            for d in _descs(0):
                d.start()

        for d in d1:
            d.wait()
        _scat(1)

        @pl.when(2 * i + 3 < ng)
        def _():
            _load(2 * i + 3, 1)

        return carry

    lax.fori_loop(0, n2, _pair, 0)
    if ng % 2:
        for d in _descs(0):
            d.wait()
        _scat(0)


def _agg1_body(src2_hbm, dst2_hbm, u_hbm, z8_hbm, p1_hbm,
               idxs, idxd, rows, acc, sem0, sem1):
    c = lax.axis_index("c")
    s = lax.axis_index("s")
    wid = s * NC + c

    @pl.when(jnp.logical_and(s == 0, c == 0))
    def _():
        pltpu.sync_copy(u_hbm, acc)

    @pl.when(jnp.logical_and(s == 0, c != 0))
    def _():
        pltpu.sync_copy(z8_hbm, acc)

    plsc.subcore_barrier()
    rows_per = E_ROWS // NW
    _pipe_gather_scatter(u_hbm, src2_hbm, dst2_hbm, wid * rows_per,
                         rows_per // KG1, KG1, idxs, idxd, rows, acc,
                         (sem0, sem1))
    plsc.subcore_barrier()

    @pl.when(s == 0)
    def _():
        pltpu.sync_copy(acc, p1_hbm.at[c])


_agg1 = functools.partial(
    pl.kernel,
    out_type=jax.ShapeDtypeStruct((NC, NP, 8), jnp.float32),
    mesh=_mesh(),
    compiler_params=pltpu.CompilerParams(needs_layout_passes=False, use_tc_tiling_on_sc=False),
    scratch_types=[
        pltpu.VMEM((2, KG1, EROW), jnp.int32),
        pltpu.VMEM((2, KG1, EROW), jnp.int32),
        pltpu.VMEM((2, KG1, EROW, 8), jnp.float32),
        pltpu.VMEM_SHARED((NP, 8), jnp.float32),
        pltpu.SemaphoreType.DMA,
        pltpu.SemaphoreType.DMA,
    ],
)(_agg1_body)


def _dense1_body(p1f_hbm, dinv_hbm, w1t_hbm, b1_hbm, g1_hbm,
                 s8b, tmp8, dinvb, gbuf, wb, bb):
    c = lax.axis_index("c")
    s = lax.axis_index("s")
    wid = s * NC + c
    pltpu.sync_copy(w1t_hbm, wb)
    pltpu.sync_copy(b1_hbm, bb)
    wv = [[wb[k, pl.ds(16 * q, L)] for q in range(4)] for k in range(3)]
    bv = [bb[pl.ds(16 * q, L)] for q in range(4)]

    for half in range(2):
        nbase = wid * NPT + half * NSC
        pltpu.sync_copy(p1f_hbm.at[0, pl.ds(8 * nbase, 8 * NSC)], s8b)
        pltpu.sync_copy(p1f_hbm.at[1, pl.ds(8 * nbase, 8 * NSC)], tmp8)
        pltpu.sync_copy(dinv_hbm.at[pl.ds(nbase, NSC)], dinvb)

        def _add(g, carry):
            s8b[pl.ds(g * L, L)] = s8b[pl.ds(g * L, L)] + tmp8[pl.ds(g * L, L)]
            return carry

        lax.fori_loop(0, NSC * 8 // L, _add, 0)

        def _node16(i, carry):
            dv16 = dinvb[pl.ds(i * L, L)]
            rowv = [s8b[pl.ds(i * 8 * L + m * L, L)] for m in range(8)]
            for j in range(L):
                r = rowv[j // 2]
                o = (j % 2) * 8
                dv = dv16[j]
                t0 = jnp.full((L,), dv * r[o])
                t1 = jnp.full((L,), dv * r[o + 1])
                t2 = jnp.full((L,), dv * r[o + 2])
                dvb = jnp.full((L,), dv)
                for q in range(4):
                    h = t0 * wv[0][q] + t1 * wv[1][q] + t2 * wv[2][q] + bv[q]
                    gbuf[q, pl.ds((i * L + j) * L, L)] = dvb * jnp.maximum(h, 0.0)
            return carry

        lax.fori_loop(0, NSC // L, _node16, 0)
        for q in range(4):
            pltpu.sync_copy(
                gbuf.at[q],
                g1_hbm.at[pl.ds(16 * (q * NP + nbase), 16 * NSC)])


_dense1 = functools.partial(
    pl.kernel,
    out_type=jax.ShapeDtypeStruct((4 * NP * 16,), jnp.float32),
    mesh=_mesh(),
    compiler_params=pltpu.CompilerParams(needs_layout_passes=False, use_tc_tiling_on_sc=False),
    scratch_types=[
        pltpu.VMEM((NSC * 8,), jnp.float32),
        pltpu.VMEM((NSC * 8,), jnp.float32),
        pltpu.VMEM((NSC,), jnp.float32),
        pltpu.VMEM((4, NSC * 16), jnp.float32),
        pltpu.VMEM((3, H), jnp.float32),
        pltpu.VMEM((H,), jnp.float32),
    ],
)(_dense1_body)


def _agg2_body(src4_hbm, dst2_hbm, g1f_hbm, agg2_hbm,
               idxs, idxd, rows, acc, sem0, sem1):
    c = lax.axis_index("c")
    s = lax.axis_index("s")
    rows_per = E_ROWS // NS

    for p in range(2):
        @pl.when(s == 0)
        def _(p=p):
            q = 2 * p + c
            pltpu.sync_copy(g1f_hbm.at[pl.ds(q * NP, NP)], acc)

        plsc.subcore_barrier()
        _pipe_gather_scatter(g1f_hbm, src4_hbm.at[2 * p + c], dst2_hbm,
                             s * rows_per, rows_per // KG, KG,
                             idxs, idxd, rows, acc, (sem0, sem1))
        plsc.subcore_barrier()

        @pl.when(s == 0)
        def _():
            pltpu.sync_copy(acc, agg2_hbm.at[p, c])

        plsc.subcore_barrier()


_agg2 = functools.partial(
    pl.kernel,
    out_type=jax.ShapeDtypeStruct((2, NC, NP, 16), jnp.float32),
    mesh=_mesh(),
    compiler_params=pltpu.CompilerParams(needs_layout_passes=False, use_tc_tiling_on_sc=False),
    scratch_types=[
        pltpu.VMEM((2, KG, EROW), jnp.int32),
        pltpu.VMEM((2, KG, EROW), jnp.int32),
        pltpu.VMEM((2, KG, EROW, 16), jnp.float32),
        pltpu.VMEM_SHARED((NP, 16), jnp.float32),
        pltpu.SemaphoreType.DMA,
        pltpu.SemaphoreType.DMA,
    ],
)(_agg2_body)


def _pq_body(agg4_hbm, dinv_hbm, m2_hbm, p_hbm, q_hbm,
             st, dinvb, pbuf, qbuf, m2b):
    c = lax.axis_index("c")
    s = lax.axis_index("s")
    wid = s * NC + c
    nbase = wid * NPT
    pltpu.sync_copy(m2_hbm, m2b)
    m2p = [m2b[0, pl.ds(16 * qi, L)] for qi in range(4)]
    m2q = [m2b[1, pl.ds(16 * qi, L)] for qi in range(4)]
    for qi in range(4):
        pltpu.sync_copy(agg4_hbm.at[qi, pl.ds(16 * nbase, 16 * NPT)],
                        st.at[qi])
    pltpu.sync_copy(dinv_hbm.at[pl.ds(nbase, NPT)], dinvb)
    lane = lax.iota(jnp.int32, L)

    def _blk(i, carry):
        dv16 = dinvb[pl.ds(i * L, L)]
        pvec = jnp.zeros((L,), jnp.float32)
        qvec = jnp.zeros((L,), jnp.float32)
        for j in range(L):
            dvb = jnp.full((L,), dv16[j])
            off = (i * L + j) * L
            accp = accq = None
            for qi in range(4):
                pre = dvb * st[qi, pl.ds(off, L)]
                tp = pre * m2p[qi]
                tq = pre * m2q[qi]
                accp = tp if accp is None else accp + tp
                accq = tq if accq is None else accq + tq
            pn = jnp.sum(accp)
            qn = jnp.sum(accq)
            m = lane == j
            pvec = jnp.where(m, jnp.full((L,), pn), pvec)
            qvec = jnp.where(m, jnp.full((L,), qn), qvec)
        pbuf[pl.ds(i * L, L)] = pvec
        qbuf[pl.ds(i * L, L)] = qvec
        return carry

    lax.fori_loop(0, NPT // L, _blk, 0)
    pltpu.sync_copy(pbuf, p_hbm.at[pl.ds(nbase, NPT)])
    pltpu.sync_copy(qbuf, q_hbm.at[pl.ds(nbase, NPT)])


_pq = functools.partial(
    pl.kernel,
    out_type=(jax.ShapeDtypeStruct((NP,), jnp.float32),
              jax.ShapeDtypeStruct((NP,), jnp.float32)),
    mesh=_mesh(),
    compiler_params=pltpu.CompilerParams(needs_layout_passes=False, use_tc_tiling_on_sc=False),
    scratch_types=[
        pltpu.VMEM((4, NPT * 16), jnp.float32),
        pltpu.VMEM((NPT,), jnp.float32),
        pltpu.VMEM((NPT,), jnp.float32),
        pltpu.VMEM((NPT,), jnp.float32),
        pltpu.VMEM((2, H), jnp.float32),
    ],
)(_pq_body)


CH = 2000
NCHUNK = N_EDGES // CH


def _edge_body(src_hbm, dst_hbm, ea0_hbm, ea1_hbm, p_hbm, q_hbm, cst_hbm,
               out_hbm, p_v, q_v, sbuf, dbuf, e0, e1, ob, cst_v):
    c = lax.axis_index("c")
    s = lax.axis_index("s")
    wid = s * NC + c
    pltpu.sync_copy(p_hbm, p_v)
    pltpu.sync_copy(q_hbm, q_v)
    pltpu.sync_copy(cst_hbm, cst_v)
    w0 = cst_v[0, :]
    w1 = cst_v[1, :]
    bcv = cst_v[2, :]

    def _chunk(j, carry):
        base = (wid + j * NW) * CH
        pltpu.sync_copy(src_hbm.at[pl.ds(base, CH)], sbuf)
        pltpu.sync_copy(dst_hbm.at[pl.ds(base, CH)], dbuf)
        pltpu.sync_copy(ea0_hbm.at[pl.ds(base, CH)], e0)
        pltpu.sync_copy(ea1_hbm.at[pl.ds(base, CH)], e1)

        def _grp(g, inner):
            off = g * L
            si = sbuf[pl.ds(off, L)]
            di = dbuf[pl.ds(off, L)]
            pv = plsc.load_gather(p_v, [si])
            qv = plsc.load_gather(q_v, [di])
            z = pv + qv + w0 * e0[pl.ds(off, L)] + w1 * e1[pl.ds(off, L)] + bcv
            ob[pl.ds(off, L)] = 1.0 / (1.0 + jnp.exp(-z))
            return inner

        lax.fori_loop(0, CH // L, _grp, 0)
        pltpu.sync_copy(ob, out_hbm.at[pl.ds(base, CH)])
        return carry

    nmine = (NCHUNK - 1 - wid) // NW + 1
    lax.fori_loop(0, nmine, _chunk, 0)


_edges = functools.partial(
    pl.kernel,
    out_type=jax.ShapeDtypeStruct((N_EDGES,), jnp.float32),
    mesh=_mesh(),
    compiler_params=pltpu.CompilerParams(needs_layout_passes=False, use_tc_tiling_on_sc=False),
    scratch_types=[
        pltpu.VMEM((NP,), jnp.float32),
        pltpu.VMEM((NP,), jnp.float32),
        pltpu.VMEM((CH,), jnp.int32),
        pltpu.VMEM((CH,), jnp.int32),
        pltpu.VMEM((CH,), jnp.float32),
        pltpu.VMEM((CH,), jnp.float32),
        pltpu.VMEM((CH,), jnp.float32),
        pltpu.VMEM((3, L), jnp.float32),
    ],
)(_edge_body)


def kernel(x, edge_index, edge_attr, W1, b1, W2, b2, Wc, bc):
    src = edge_index[0]
    dst = edge_index[1]
    padi = jnp.full((E_PAD - N_EDGES,), NP - 1, jnp.int32)
    src2 = jnp.concatenate([src, padi]).reshape(E_ROWS, EROW)
    dst2 = jnp.concatenate([dst, padi]).reshape(E_ROWS, EROW)
    src4 = jnp.stack([src2, src2 + NP, src2 + 2 * NP, src2 + 3 * NP])
    xcols = [jnp.pad(x[:, k], (0, NP - N_NODES)) for k in range(3)]
    z8 = jnp.zeros((NP, 8), jnp.float32)

    degp = _deg(dst2)
    dinvv, uf = _norm(degp, *xcols)
    u2d = uf.reshape(NP, 8)
    p1 = _agg1(src2, dst2, u2d, z8)
    p1f = p1.reshape(NC, NP * 8)
    g1fl = _dense1(p1f, dinvv, W1.T, b1)
    g1f = g1fl.reshape(4 * NP, 16)
    agg4 = _agg2(src4, dst2, g1f).reshape(4, NP * 16)
    wsd = Wc[0, :2 * H].reshape(2, H)
    m2 = jnp.dot(wsd, W2, precision=_PREC)
    pvec, qvec = _pq(agg4, dinvv, m2)
    bconst = bc[0] + jnp.dot(b2, wsd[0]) + jnp.dot(b2, wsd[1])
    cst = jnp.stack([
        jnp.full((L,), Wc[0, 2 * H], jnp.float32),
        jnp.full((L,), Wc[0, 2 * H + 1], jnp.float32),
        jnp.full((L,), bconst, jnp.float32),
    ])
    out = _edges(src, dst, edge_attr[:, 0], edge_attr[:, 1],
                 pvec, qvec, cst)
    return out[:, None]


# speedup vs baseline: 1.0371x; 1.0321x over previous
import functools

import jax
import jax.numpy as jnp
from jax import lax
from jax.experimental import pallas as pl
from jax.experimental.pallas import tpu as pltpu
from jax.experimental.pallas import tpu_sc as plsc

N_NODES = 50000
N_EDGES = 800000
H = 64

NP = 50176
NC, NS, L = 2, 16, 16
NW = NC * NS
EROW = 128
E_ROWS = 6400
E_PAD = E_ROWS * EROW
KG = 8
KG1 = 8
R = 3136
GR = NP // R

_PREC = lax.Precision.HIGHEST
NPT = NP // NW
NSC = NPT // 2


def _mesh():
    return plsc.VectorSubcoreMesh(core_axis_name="c", subcore_axis_name="s")


def _deg_body(dst2_hbm, out_hbm, hist, idxb):
    c = lax.axis_index("c")
    s = lax.axis_index("s")
    wid = s * NC + c
    zv = jnp.zeros((L,), jnp.float32)

    def _z(i, carry):
        hist[pl.ds(i * L, L)] = zv
        return carry

    lax.fori_loop(0, NP // L, _z, 0)

    ones = jnp.ones((L,), jnp.float32)
    rows_per = E_ROWS // NW

    def _outer(o, carry):
        rb = wid * rows_per + o * KG
        pltpu.sync_copy(dst2_hbm.at[pl.ds(rb, KG)], idxb)
        for k in range(KG):
            for g in range(EROW // L):
                idx = idxb[k, pl.ds(g * L, L)]
                plsc.addupdate_scatter(hist, [idx], ones)
        return carry

    lax.fori_loop(0, rows_per // KG, _outer, 0)
    pltpu.sync_copy(hist, out_hbm.at[wid, 0])


_deg = functools.partial(
    pl.kernel,
    out_type=jax.ShapeDtypeStruct((NW, 1, NP), jnp.float32),
    mesh=_mesh(),
    compiler_params=pltpu.CompilerParams(needs_layout_passes=False, use_tc_tiling_on_sc=False),
    scratch_types=[
        pltpu.VMEM((NP,), jnp.float32),
        pltpu.VMEM((KG, EROW), jnp.int32),
    ],
)(_deg_body)


def _vrsqrt(d):
    magic = jnp.full((L,), 0x5F3759DF, jnp.int32)
    y = plsc.bitcast(magic - (plsc.bitcast(d, jnp.int32) >> 1), jnp.float32)
    for _ in range(4):
        y = y * (1.5 - 0.5 * d * y * y)
    return y


def _norm_body(degp_hbm, x0_hbm, x1_hbm, x2_hbm, dinv_hbm, uf_hbm,
               pall, dinvb, ubf, c0b, c1b, c2b):
    c = lax.axis_index("c")
    s = lax.axis_index("s")
    wid = s * NC + c
    nbase = wid * NPT
    pltpu.sync_copy(degp_hbm.at[:, 0, pl.ds(nbase, NPT)], pall)

    def _r(g, carry):
        d = pall[0, pl.ds(g * L, L)]
        for w in range(1, NW):
            d = d + pall[w, pl.ds(g * L, L)]
        dinvb[pl.ds(g * L, L)] = _vrsqrt(d + 1.0)
        return carry

    lax.fori_loop(0, NPT // L, _r, 0)

    pltpu.sync_copy(x0_hbm.at[pl.ds(nbase, NPT)], c0b)
    pltpu.sync_copy(x1_hbm.at[pl.ds(nbase, NPT)], c1b)
    pltpu.sync_copy(x2_hbm.at[pl.ds(nbase, NPT)], c2b)
    lane = lax.iota(jnp.int32, L)

    def _u(g, carry):
        v = dinvb[pl.ds(g * L, L)]
        x0 = c0b[pl.ds(g * L, L)]
        x1 = c1b[pl.ds(g * L, L)]
        x2 = c2b[pl.ds(g * L, L)]
        for j in range(8):
            na, nb = 2 * j, 2 * j + 1
            z = jnp.zeros((L,), jnp.float32)
            z = jnp.where(lane == 0, jnp.full((L,), v[na] * x0[na]), z)
            z = jnp.where(lane == 1, jnp.full((L,), v[na] * x1[na]), z)
            z = jnp.where(lane == 2, jnp.full((L,), v[na] * x2[na]), z)
            z = jnp.where(lane == 8, jnp.full((L,), v[nb] * x0[nb]), z)
            z = jnp.where(lane == 9, jnp.full((L,), v[nb] * x1[nb]), z)
            z = jnp.where(lane == 10, jnp.full((L,), v[nb] * x2[nb]), z)
            ubf[pl.ds(g * 8 * L + j * L, L)] = z
        return carry

    lax.fori_loop(0, NPT // L, _u, 0)
    pltpu.sync_copy(dinvb, dinv_hbm.at[pl.ds(nbase, NPT)])
    pltpu.sync_copy(ubf, uf_hbm.at[pl.ds(8 * nbase, 8 * NPT)])


_norm = functools.partial(
    pl.kernel,
    out_type=(jax.ShapeDtypeStruct((NP,), jnp.float32),
              jax.ShapeDtypeStruct((NP * 8,), jnp.float32)),
    mesh=_mesh(),
    compiler_params=pltpu.CompilerParams(needs_layout_passes=False, use_tc_tiling_on_sc=False),
    scratch_types=[
        pltpu.VMEM((NW, NPT), jnp.float32),
        pltpu.VMEM((NPT,), jnp.float32),
        pltpu.VMEM((NPT * 8,), jnp.float32),
        pltpu.VMEM((NPT,), jnp.float32),
        pltpu.VMEM((NPT,), jnp.float32),
        pltpu.VMEM((NPT,), jnp.float32),
    ],
)(_norm_body)


def _pipe_gather_scatter(tbl_hbm, si_hbm, di_hbm, base, ng, kg,
                         idxs, idxd, rows, acc, sems):
    def _load(g, b):
        pltpu.sync_copy(si_hbm.at[pl.ds(base + g * kg, kg)], idxs.at[b])
        pltpu.sync_copy(di_hbm.at[pl.ds(base + g * kg, kg)], idxd.at[b])

    def _descs(b):
        return [
            pltpu.make_async_copy(tbl_hbm.at[idxs.at[b, k]], rows.at[b, k],
                                  sems[b])
            for k in range(kg)
        ]

    def _scat(b):
        for k in range(kg):
            pltpu.sync_copy(rows.at[b, k], acc.at[idxd.at[b, k]], add=True)

    _load(0, 0)
    for d in _descs(0):
        d.start()
    _load(1, 1)
    n2 = ng // 2

    def _pair(i, carry):
        d1 = _descs(1)
        for d in d1:
            d.start()
        d0 = _descs(0)
        for d in d0:
            d.wait()
        _scat(0)

        @pl.when(2 * i + 2 < ng)
        def _():
            _load(2 * i + 2, 0)
            for d in _descs(0):
                d.start()

        for d in d1:
            d.wait()
        _scat(1)

        @pl.when(2 * i + 3 < ng)
        def _():
            _load(2 * i + 3, 1)

        return carry

    lax.fori_loop(0, n2, _pair, 0)
    if ng % 2:
        for d in _descs(0):
            d.wait()
        _scat(0)


def _agg1_body(src2_hbm, dst2_hbm, u_hbm, z8_hbm, p1_hbm,
               idxs, idxd, rows, acc, sem0, sem1):
    c = lax.axis_index("c")
    s = lax.axis_index("s")
    wid = s * NC + c

    @pl.when(jnp.logical_and(s == 0, c == 0))
    def _():
        pltpu.sync_copy(u_hbm, acc)

    @pl.when(jnp.logical_and(s == 0, c != 0))
    def _():
        pltpu.sync_copy(z8_hbm, acc)

    plsc.subcore_barrier()
    rows_per = E_ROWS // NW
    _pipe_gather_scatter(u_hbm, src2_hbm, dst2_hbm, wid * rows_per,
                         rows_per // KG1, KG1, idxs, idxd, rows, acc,
                         (sem0, sem1))
    plsc.subcore_barrier()

    @pl.when(s == 0)
    def _():
        pltpu.sync_copy(acc, p1_hbm.at[c])


_agg1 = functools.partial(
    pl.kernel,
    out_type=jax.ShapeDtypeStruct((NC, NP, 8), jnp.float32),
    mesh=_mesh(),
    compiler_params=pltpu.CompilerParams(needs_layout_passes=False, use_tc_tiling_on_sc=False),
    scratch_types=[
        pltpu.VMEM((2, KG1, EROW), jnp.int32),
        pltpu.VMEM((2, KG1, EROW), jnp.int32),
        pltpu.VMEM((2, KG1, EROW, 8), jnp.float32),
        pltpu.VMEM_SHARED((NP, 8), jnp.float32),
        pltpu.SemaphoreType.DMA,
        pltpu.SemaphoreType.DMA,
    ],
)(_agg1_body)


def _dense1_body(p1f_hbm, dinv_hbm, w1t_hbm, b1_hbm, g1_hbm,
                 s8b, tmp8, dinvb, gbuf, wb, bb):
    c = lax.axis_index("c")
    s = lax.axis_index("s")
    wid = s * NC + c
    pltpu.sync_copy(w1t_hbm, wb)
    pltpu.sync_copy(b1_hbm, bb)
    wv = [[wb[k, pl.ds(16 * q, L)] for q in range(4)] for k in range(3)]
    bv = [bb[pl.ds(16 * q, L)] for q in range(4)]

    for half in range(2):
        nbase = wid * NPT + half * NSC
        pltpu.sync_copy(p1f_hbm.at[0, pl.ds(8 * nbase, 8 * NSC)], s8b)
        pltpu.sync_copy(p1f_hbm.at[1, pl.ds(8 * nbase, 8 * NSC)], tmp8)
        pltpu.sync_copy(dinv_hbm.at[pl.ds(nbase, NSC)], dinvb)

        def _add(g, carry):
            s8b[pl.ds(g * L, L)] = s8b[pl.ds(g * L, L)] + tmp8[pl.ds(g * L, L)]
            return carry

        lax.fori_loop(0, NSC * 8 // L, _add, 0)

        def _node16(i, carry):
            dv16 = dinvb[pl.ds(i * L, L)]
            rowv = [s8b[pl.ds(i * 8 * L + m * L, L)] for m in range(8)]
            for j in range(L):
                r = rowv[j // 2]
                o = (j % 2) * 8
                dv = dv16[j]
                t0 = jnp.full((L,), dv * r[o])
                t1 = jnp.full((L,), dv * r[o + 1])
                t2 = jnp.full((L,), dv * r[o + 2])
                dvb = jnp.full((L,), dv)
                for q in range(4):
                    h = t0 * wv[0][q] + t1 * wv[1][q] + t2 * wv[2][q] + bv[q]
                    gbuf[q, pl.ds((i * L + j) * L, L)] = dvb * jnp.maximum(h, 0.0)
            return carry

        lax.fori_loop(0, NSC // L, _node16, 0)
        for q in range(4):
            pltpu.sync_copy(
                gbuf.at[q],
                g1_hbm.at[pl.ds(16 * (q * NP + nbase), 16 * NSC)])


_dense1 = functools.partial(
    pl.kernel,
    out_type=jax.ShapeDtypeStruct((4 * NP * 16,), jnp.float32),
    mesh=_mesh(),
    compiler_params=pltpu.CompilerParams(needs_layout_passes=False, use_tc_tiling_on_sc=False),
    scratch_types=[
        pltpu.VMEM((NSC * 8,), jnp.float32),
        pltpu.VMEM((NSC * 8,), jnp.float32),
        pltpu.VMEM((NSC,), jnp.float32),
        pltpu.VMEM((4, NSC * 16), jnp.float32),
        pltpu.VMEM((3, H), jnp.float32),
        pltpu.VMEM((H,), jnp.float32),
    ],
)(_dense1_body)


def _agg2_body(src4_hbm, dst2_hbm, g1f_hbm, agg2_hbm,
               idxs, idxd, rows, acc, sem0, sem1):
    c = lax.axis_index("c")
    s = lax.axis_index("s")
    rows_per = E_ROWS // NS

    for p in range(2):
        @pl.when(s == 0)
        def _(p=p):
            q = 2 * p + c
            pltpu.sync_copy(g1f_hbm.at[pl.ds(q * NP, NP)], acc)

        plsc.subcore_barrier()
        _pipe_gather_scatter(g1f_hbm, src4_hbm.at[2 * p + c], dst2_hbm,
                             s * rows_per, rows_per // KG, KG,
                             idxs, idxd, rows, acc, (sem0, sem1))
        plsc.subcore_barrier()

        @pl.when(s == 0)
        def _():
            pltpu.sync_copy(acc, agg2_hbm.at[p, c])

        plsc.subcore_barrier()


_agg2 = functools.partial(
    pl.kernel,
    out_type=jax.ShapeDtypeStruct((2, NC, NP, 16), jnp.float32),
    mesh=_mesh(),
    compiler_params=pltpu.CompilerParams(needs_layout_passes=False, use_tc_tiling_on_sc=False),
    scratch_types=[
        pltpu.VMEM((2, KG, EROW), jnp.int32),
        pltpu.VMEM((2, KG, EROW), jnp.int32),
        pltpu.VMEM((2, KG, EROW, 16), jnp.float32),
        pltpu.VMEM_SHARED((NP, 16), jnp.float32),
        pltpu.SemaphoreType.DMA,
        pltpu.SemaphoreType.DMA,
    ],
)(_agg2_body)


def _pq_body(agg4_hbm, dinv_hbm, m2_hbm, p_hbm, q_hbm,
             st, dinvb, pbuf, qbuf, m2b):
    c = lax.axis_index("c")
    s = lax.axis_index("s")
    wid = s * NC + c
    nbase = wid * NPT
    pltpu.sync_copy(m2_hbm, m2b)
    m2p = [m2b[0, pl.ds(16 * qi, L)] for qi in range(4)]
    m2q = [m2b[1, pl.ds(16 * qi, L)] for qi in range(4)]
    for qi in range(4):
        pltpu.sync_copy(agg4_hbm.at[qi, pl.ds(16 * nbase, 16 * NPT)],
                        st.at[qi])
    pltpu.sync_copy(dinv_hbm.at[pl.ds(nbase, NPT)], dinvb)
    lane = lax.iota(jnp.int32, L)

    def _blk(i, carry):
        dv16 = dinvb[pl.ds(i * L, L)]
        pvec = jnp.zeros((L,), jnp.float32)
        qvec = jnp.zeros((L,), jnp.float32)
        for j in range(L):
            dvb = jnp.full((L,), dv16[j])
            off = (i * L + j) * L
            accp = accq = None
            for qi in range(4):
                pre = dvb * st[qi, pl.ds(off, L)]
                tp = pre * m2p[qi]
                tq = pre * m2q[qi]
                accp = tp if accp is None else accp + tp
                accq = tq if accq is None else accq + tq
            pn = jnp.sum(accp)
            qn = jnp.sum(accq)
            m = lane == j
            pvec = jnp.where(m, jnp.full((L,), pn), pvec)
            qvec = jnp.where(m, jnp.full((L,), qn), qvec)
        pbuf[pl.ds(i * L, L)] = pvec
        qbuf[pl.ds(i * L, L)] = qvec
        return carry

    lax.fori_loop(0, NPT // L, _blk, 0)
    pltpu.sync_copy(pbuf, p_hbm.at[pl.ds(nbase, NPT)])
    pltpu.sync_copy(qbuf, q_hbm.at[pl.ds(nbase, NPT)])


_pq = functools.partial(
    pl.kernel,
    out_type=(jax.ShapeDtypeStruct((NP,), jnp.float32),
              jax.ShapeDtypeStruct((NP,), jnp.float32)),
    mesh=_mesh(),
    compiler_params=pltpu.CompilerParams(needs_layout_passes=False, use_tc_tiling_on_sc=False),
    scratch_types=[
        pltpu.VMEM((4, NPT * 16), jnp.float32),
        pltpu.VMEM((NPT,), jnp.float32),
        pltpu.VMEM((NPT,), jnp.float32),
        pltpu.VMEM((NPT,), jnp.float32),
        pltpu.VMEM((2, H), jnp.float32),
    ],
)(_pq_body)


CH = 2000
NCHUNK = N_EDGES // CH


def _edge_body(src_hbm, dst_hbm, ea0_hbm, ea1_hbm, p_hbm, q_hbm, cst_hbm,
               out_hbm, p_v, q_v, sbuf, dbuf, e0, e1, ob, cst_v):
    c = lax.axis_index("c")
    s = lax.axis_index("s")
    wid = s * NC + c
    pltpu.sync_copy(p_hbm, p_v)
    pltpu.sync_copy(q_hbm, q_v)
    pltpu.sync_copy(cst_hbm, cst_v)
    w0 = cst_v[0, :]
    w1 = cst_v[1, :]
    bcv = cst_v[2, :]

    def _chunk(j, carry):
        base = (wid + j * NW) * CH
        pltpu.sync_copy(src_hbm.at[pl.ds(base, CH)], sbuf)
        pltpu.sync_copy(dst_hbm.at[pl.ds(base, CH)], dbuf)
        pltpu.sync_copy(ea0_hbm.at[pl.ds(base, CH)], e0)
        pltpu.sync_copy(ea1_hbm.at[pl.ds(base, CH)], e1)

        def _grp(g, inner):
            off = g * L
            si = sbuf[pl.ds(off, L)]
            di = dbuf[pl.ds(off, L)]
            pv = plsc.load_gather(p_v, [si])
            qv = plsc.load_gather(q_v, [di])
            z = pv + qv + w0 * e0[pl.ds(off, L)] + w1 * e1[pl.ds(off, L)] + bcv
            ob[pl.ds(off, L)] = 1.0 / (1.0 + jnp.exp(-z))
            return inner

        lax.fori_loop(0, CH // L, _grp, 0)
        pltpu.sync_copy(ob, out_hbm.at[pl.ds(base, CH)])
        return carry

    nmine = (NCHUNK - 1 - wid) // NW + 1
    lax.fori_loop(0, nmine, _chunk, 0)


_edges = functools.partial(
    pl.kernel,
    out_type=jax.ShapeDtypeStruct((N_EDGES,), jnp.float32),
    mesh=_mesh(),
    compiler_params=pltpu.CompilerParams(needs_layout_passes=False, use_tc_tiling_on_sc=False),
    scratch_types=[
        pltpu.VMEM((NP,), jnp.float32),
        pltpu.VMEM((NP,), jnp.float32),
        pltpu.VMEM((CH,), jnp.int32),
        pltpu.VMEM((CH,), jnp.int32),
        pltpu.VMEM((CH,), jnp.float32),
        pltpu.VMEM((CH,), jnp.float32),
        pltpu.VMEM((CH,), jnp.float32),
        pltpu.VMEM((3, L), jnp.float32),
    ],
)(_edge_body)


def kernel(x, edge_index, edge_attr, W1, b1, W2, b2, Wc, bc):
    src = edge_index[0]
    dst = edge_index[1]
    padi = jnp.full((E_PAD - N_EDGES,), NP - 1, jnp.int32)
    src2 = jnp.concatenate([src, padi]).reshape(E_ROWS, EROW)
    dst2 = jnp.concatenate([dst, padi]).reshape(E_ROWS, EROW)
    src4 = jnp.stack([src2, src2 + NP, src2 + 2 * NP, src2 + 3 * NP])
    xcols = [jnp.pad(x[:, k], (0, NP - N_NODES)) for k in range(3)]
    z8 = jnp.zeros((NP, 8), jnp.float32)

    degp = _deg(dst2)
    dinvv, uf = _norm(degp, *xcols)
    u2d = uf.reshape(NP, 8)
    p1 = _agg1(src2, dst2, u2d, z8)
    p1f = p1.reshape(NC, NP * 8)
    g1fl = _dense1(p1f, dinvv, W1.T, b1)
    g1f = g1fl.reshape(4 * NP, 16)
    agg4 = _agg2(src4, dst2, g1f).reshape(4, NP * 16)
    wsd = Wc[0, :2 * H].reshape(2, H)
    m2 = jnp.dot(wsd, W2, precision=_PREC)
    pvec, qvec = _pq(agg4, dinvv, m2)
    bconst = bc[0] + jnp.dot(b2, wsd[0]) + jnp.dot(b2, wsd[1])
    cst = jnp.stack([
        jnp.full((L,), Wc[0, 2 * H], jnp.float32),
        jnp.full((L,), Wc[0, 2 * H + 1], jnp.float32),
        jnp.full((L,), bconst, jnp.float32),
    ])
    out = _edges(src, dst, edge_attr[:, 0], edge_attr[:, 1],
                 pvec, qvec, cst)
    return out[:, None]


# speedup vs baseline: 1.0764x; 1.0379x over previous
import functools

import jax
import jax.numpy as jnp
from jax import lax
from jax.experimental import pallas as pl
from jax.experimental.pallas import tpu as pltpu
from jax.experimental.pallas import tpu_sc as plsc

N_NODES = 50000
N_EDGES = 800000
H = 64

NP = 50176
NC, NS, L = 2, 16, 16
NW = NC * NS
EROW = 128
E_ROWS = 6400
E_PAD = E_ROWS * EROW
KG = 8
KG1 = 8
R = 3136
GR = NP // R

_PREC = lax.Precision.HIGHEST
NPT = NP // NW
NSC = NPT // 2


def _mesh():
    return plsc.VectorSubcoreMesh(core_axis_name="c", subcore_axis_name="s")


def _deg_body(dst2_hbm, out_hbm, hist, idxb):
    c = lax.axis_index("c")
    s = lax.axis_index("s")
    wid = s * NC + c
    zv = jnp.zeros((L,), jnp.float32)

    def _z(i, carry):
        hist[pl.ds(i * L, L)] = zv
        return carry

    lax.fori_loop(0, NP // L, _z, 0)

    ones = jnp.ones((L,), jnp.float32)
    rows_per = E_ROWS // NW

    def _outer(o, carry):
        rb = wid * rows_per + o * KG
        pltpu.sync_copy(dst2_hbm.at[pl.ds(rb, KG)], idxb)
        for k in range(KG):
            for g in range(EROW // L):
                idx = idxb[k, pl.ds(g * L, L)]
                plsc.addupdate_scatter(hist, [idx], ones)
        return carry

    lax.fori_loop(0, rows_per // KG, _outer, 0)
    pltpu.sync_copy(hist, out_hbm.at[wid, 0])


_deg = functools.partial(
    pl.kernel,
    out_type=jax.ShapeDtypeStruct((NW, 1, NP), jnp.float32),
    mesh=_mesh(),
    compiler_params=pltpu.CompilerParams(needs_layout_passes=False, use_tc_tiling_on_sc=False),
    scratch_types=[
        pltpu.VMEM((NP,), jnp.float32),
        pltpu.VMEM((KG, EROW), jnp.int32),
    ],
)(_deg_body)


def _vrsqrt(d):
    magic = jnp.full((L,), 0x5F3759DF, jnp.int32)
    y = plsc.bitcast(magic - (plsc.bitcast(d, jnp.int32) >> 1), jnp.float32)
    for _ in range(4):
        y = y * (1.5 - 0.5 * d * y * y)
    return y


def _norm_body(degp_hbm, x0_hbm, x1_hbm, x2_hbm, dinv_hbm, uf_hbm,
               pall, dinvb, ubf, c0b, c1b, c2b):
    c = lax.axis_index("c")
    s = lax.axis_index("s")
    wid = s * NC + c
    nbase = wid * NPT
    pltpu.sync_copy(degp_hbm.at[:, 0, pl.ds(nbase, NPT)], pall)

    def _r(g, carry):
        d = pall[0, pl.ds(g * L, L)]
        for w in range(1, NW):
            d = d + pall[w, pl.ds(g * L, L)]
        dinvb[pl.ds(g * L, L)] = _vrsqrt(d + 1.0)
        return carry

    lax.fori_loop(0, NPT // L, _r, 0)

    pltpu.sync_copy(x0_hbm.at[pl.ds(nbase, NPT)], c0b)
    pltpu.sync_copy(x1_hbm.at[pl.ds(nbase, NPT)], c1b)
    pltpu.sync_copy(x2_hbm.at[pl.ds(nbase, NPT)], c2b)
    lane = lax.iota(jnp.int32, L)

    def _u(g, carry):
        v = dinvb[pl.ds(g * L, L)]
        x0 = c0b[pl.ds(g * L, L)]
        x1 = c1b[pl.ds(g * L, L)]
        x2 = c2b[pl.ds(g * L, L)]
        for j in range(8):
            na, nb = 2 * j, 2 * j + 1
            z = jnp.zeros((L,), jnp.float32)
            z = jnp.where(lane == 0, jnp.full((L,), v[na] * x0[na]), z)
            z = jnp.where(lane == 1, jnp.full((L,), v[na] * x1[na]), z)
            z = jnp.where(lane == 2, jnp.full((L,), v[na] * x2[na]), z)
            z = jnp.where(lane == 8, jnp.full((L,), v[nb] * x0[nb]), z)
            z = jnp.where(lane == 9, jnp.full((L,), v[nb] * x1[nb]), z)
            z = jnp.where(lane == 10, jnp.full((L,), v[nb] * x2[nb]), z)
            ubf[pl.ds(g * 8 * L + j * L, L)] = z
        return carry

    lax.fori_loop(0, NPT // L, _u, 0)
    pltpu.sync_copy(dinvb, dinv_hbm.at[pl.ds(nbase, NPT)])
    pltpu.sync_copy(ubf, uf_hbm.at[pl.ds(8 * nbase, 8 * NPT)])


_norm = functools.partial(
    pl.kernel,
    out_type=(jax.ShapeDtypeStruct((NP,), jnp.float32),
              jax.ShapeDtypeStruct((NP * 8,), jnp.float32)),
    mesh=_mesh(),
    compiler_params=pltpu.CompilerParams(needs_layout_passes=False, use_tc_tiling_on_sc=False),
    scratch_types=[
        pltpu.VMEM((NW, NPT), jnp.float32),
        pltpu.VMEM((NPT,), jnp.float32),
        pltpu.VMEM((NPT * 8,), jnp.float32),
        pltpu.VMEM((NPT,), jnp.float32),
        pltpu.VMEM((NPT,), jnp.float32),
        pltpu.VMEM((NPT,), jnp.float32),
    ],
)(_norm_body)


def _pipe_gather_scatter(tbl_hbm, si_hbm, di_hbm, base, ng, kg,
                         idxs, idxd, rows, acc, sems):
    def _load(g, b):
        pltpu.sync_copy(si_hbm.at[pl.ds(base + g * kg, kg)], idxs.at[b])
        pltpu.sync_copy(di_hbm.at[pl.ds(base + g * kg, kg)], idxd.at[b])

    def _descs(b):
        return [
            pltpu.make_async_copy(tbl_hbm.at[idxs.at[b, k]], rows.at[b, k],
                                  sems[b])
            for k in range(kg)
        ]

    def _scat(b):
        for k in range(kg):
            pltpu.sync_copy(rows.at[b, k], acc.at[idxd.at[b, k]], add=True)

    _load(0, 0)
    for d in _descs(0):
        d.start()
    _load(1, 1)
    n2 = ng // 2

    def _pair(i, carry):
        d1 = _descs(1)
        for d in d1:
            d.start()
        d0 = _descs(0)
        for d in d0:
            d.wait()
        _scat(0)

        @pl.when(2 * i + 2 < ng)
        def _():
            _load(2 * i + 2, 0)
            for d in _descs(0):
                d.start()

        for d in d1:
            d.wait()
        _scat(1)

        @pl.when(2 * i + 3 < ng)
        def _():
            _load(2 * i + 3, 1)

        return carry

    lax.fori_loop(0, n2, _pair, 0)
    if ng % 2:
        for d in _descs(0):
            d.wait()
        _scat(0)


def _agg1_body(src2_hbm, dst2_hbm, u_hbm, z8_hbm, p1_hbm,
               idxs, idxd, rows, acc, sem0, sem1):
    c = lax.axis_index("c")
    s = lax.axis_index("s")
    wid = s * NC + c

    @pl.when(jnp.logical_and(s == 0, c == 0))
    def _():
        pltpu.sync_copy(u_hbm, acc)

    @pl.when(jnp.logical_and(s == 0, c != 0))
    def _():
        pltpu.sync_copy(z8_hbm, acc)

    plsc.subcore_barrier()
    rows_per = E_ROWS // NW
    _pipe_gather_scatter(u_hbm, src2_hbm, dst2_hbm, wid * rows_per,
                         rows_per // KG1, KG1, idxs, idxd, rows, acc,
                         (sem0, sem1))
    plsc.subcore_barrier()

    @pl.when(s == 0)
    def _():
        pltpu.sync_copy(acc, p1_hbm.at[c])


_agg1 = functools.partial(
    pl.kernel,
    out_type=jax.ShapeDtypeStruct((NC, NP, 8), jnp.float32),
    mesh=_mesh(),
    compiler_params=pltpu.CompilerParams(needs_layout_passes=False, use_tc_tiling_on_sc=False),
    scratch_types=[
        pltpu.VMEM((2, KG1, EROW), jnp.int32),
        pltpu.VMEM((2, KG1, EROW), jnp.int32),
        pltpu.VMEM((2, KG1, EROW, 8), jnp.float32),
        pltpu.VMEM_SHARED((NP, 8), jnp.float32),
        pltpu.SemaphoreType.DMA,
        pltpu.SemaphoreType.DMA,
    ],
)(_agg1_body)


def _dense1_body(p1f_hbm, dinv_hbm, w1t_hbm, b1_hbm, g1_hbm,
                 s8b, tmp8, dinvb, gbuf, wb, bb):
    c = lax.axis_index("c")
    s = lax.axis_index("s")
    wid = s * NC + c
    pltpu.sync_copy(w1t_hbm, wb)
    pltpu.sync_copy(b1_hbm, bb)
    wv = [[wb[k, pl.ds(16 * q, L)] for q in range(4)] for k in range(3)]
    bv = [bb[pl.ds(16 * q, L)] for q in range(4)]

    for half in range(2):
        nbase = wid * NPT + half * NSC
        pltpu.sync_copy(p1f_hbm.at[0, pl.ds(8 * nbase, 8 * NSC)], s8b)
        pltpu.sync_copy(p1f_hbm.at[1, pl.ds(8 * nbase, 8 * NSC)], tmp8)
        pltpu.sync_copy(dinv_hbm.at[pl.ds(nbase, NSC)], dinvb)

        def _add(g, carry):
            s8b[pl.ds(g * L, L)] = s8b[pl.ds(g * L, L)] + tmp8[pl.ds(g * L, L)]
            return carry

        lax.fori_loop(0, NSC * 8 // L, _add, 0)

        def _node16(i, carry):
            dv16 = dinvb[pl.ds(i * L, L)]
            rowv = [s8b[pl.ds(i * 8 * L + m * L, L)] for m in range(8)]
            for j in range(L):
                r = rowv[j // 2]
                o = (j % 2) * 8
                dv = dv16[j]
                t0 = jnp.full((L,), dv * r[o])
                t1 = jnp.full((L,), dv * r[o + 1])
                t2 = jnp.full((L,), dv * r[o + 2])
                dvb = jnp.full((L,), dv)
                for q in range(4):
                    h = t0 * wv[0][q] + t1 * wv[1][q] + t2 * wv[2][q] + bv[q]
                    gbuf[q, pl.ds((i * L + j) * L, L)] = dvb * jnp.maximum(h, 0.0)
            return carry

        lax.fori_loop(0, NSC // L, _node16, 0)
        for q in range(4):
            pltpu.sync_copy(
                gbuf.at[q],
                g1_hbm.at[pl.ds(16 * (q * NP + nbase), 16 * NSC)])


_dense1 = functools.partial(
    pl.kernel,
    out_type=jax.ShapeDtypeStruct((4 * NP * 16,), jnp.float32),
    mesh=_mesh(),
    compiler_params=pltpu.CompilerParams(needs_layout_passes=False, use_tc_tiling_on_sc=False),
    scratch_types=[
        pltpu.VMEM((NSC * 8,), jnp.float32),
        pltpu.VMEM((NSC * 8,), jnp.float32),
        pltpu.VMEM((NSC,), jnp.float32),
        pltpu.VMEM((4, NSC * 16), jnp.float32),
        pltpu.VMEM((3, H), jnp.float32),
        pltpu.VMEM((H,), jnp.float32),
    ],
)(_dense1_body)


def _agg2_body(src4_hbm, dst2_hbm, g1f_hbm, agg2_hbm,
               idxs, idxd, rows, acc, sem0, sem1):
    c = lax.axis_index("c")
    s = lax.axis_index("s")
    rows_per = E_ROWS // NS

    for p in range(2):
        @pl.when(s == 0)
        def _(p=p):
            q = 2 * p + c
            pltpu.sync_copy(g1f_hbm.at[pl.ds(q * NP, NP)], acc)

        plsc.subcore_barrier()
        _pipe_gather_scatter(g1f_hbm, src4_hbm.at[2 * p + c], dst2_hbm,
                             s * rows_per, rows_per // KG, KG,
                             idxs, idxd, rows, acc, (sem0, sem1))
        plsc.subcore_barrier()

        @pl.when(s == 0)
        def _():
            pltpu.sync_copy(acc, agg2_hbm.at[p, c])

        plsc.subcore_barrier()


_agg2 = functools.partial(
    pl.kernel,
    out_type=jax.ShapeDtypeStruct((2, NC, NP, 16), jnp.float32),
    mesh=_mesh(),
    compiler_params=pltpu.CompilerParams(needs_layout_passes=False, use_tc_tiling_on_sc=False),
    scratch_types=[
        pltpu.VMEM((2, KG, EROW), jnp.int32),
        pltpu.VMEM((2, KG, EROW), jnp.int32),
        pltpu.VMEM((2, KG, EROW, 16), jnp.float32),
        pltpu.VMEM_SHARED((NP, 16), jnp.float32),
        pltpu.SemaphoreType.DMA,
        pltpu.SemaphoreType.DMA,
    ],
)(_agg2_body)


def _pq_body(agg4_hbm, dinv_hbm, m2_hbm, p_hbm, q_hbm,
             st, dinvb, pbuf, qbuf, m2b):
    c = lax.axis_index("c")
    s = lax.axis_index("s")
    wid = s * NC + c
    nbase = wid * NPT
    pltpu.sync_copy(m2_hbm, m2b)
    m2p = [m2b[0, pl.ds(16 * qi, L)] for qi in range(4)]
    m2q = [m2b[1, pl.ds(16 * qi, L)] for qi in range(4)]
    for qi in range(4):
        pltpu.sync_copy(agg4_hbm.at[qi, pl.ds(16 * nbase, 16 * NPT)],
                        st.at[qi])
    pltpu.sync_copy(dinv_hbm.at[pl.ds(nbase, NPT)], dinvb)
    lane = lax.iota(jnp.int32, L)

    def _blk(i, carry):
        dv16 = dinvb[pl.ds(i * L, L)]
        pvec = jnp.zeros((L,), jnp.float32)
        qvec = jnp.zeros((L,), jnp.float32)
        for j in range(L):
            dvb = jnp.full((L,), dv16[j])
            off = (i * L + j) * L
            accp = accq = None
            for qi in range(4):
                pre = dvb * st[qi, pl.ds(off, L)]
                tp = pre * m2p[qi]
                tq = pre * m2q[qi]
                accp = tp if accp is None else accp + tp
                accq = tq if accq is None else accq + tq
            pn = jnp.sum(accp)
            qn = jnp.sum(accq)
            m = lane == j
            pvec = jnp.where(m, jnp.full((L,), pn), pvec)
            qvec = jnp.where(m, jnp.full((L,), qn), qvec)
        pbuf[pl.ds(i * L, L)] = pvec
        qbuf[pl.ds(i * L, L)] = qvec
        return carry

    lax.fori_loop(0, NPT // L, _blk, 0)
    pltpu.sync_copy(pbuf, p_hbm.at[pl.ds(nbase, NPT)])
    pltpu.sync_copy(qbuf, q_hbm.at[pl.ds(nbase, NPT)])


_pq = functools.partial(
    pl.kernel,
    out_type=(jax.ShapeDtypeStruct((NP,), jnp.float32),
              jax.ShapeDtypeStruct((NP,), jnp.float32)),
    mesh=_mesh(),
    compiler_params=pltpu.CompilerParams(needs_layout_passes=False, use_tc_tiling_on_sc=False),
    scratch_types=[
        pltpu.VMEM((4, NPT * 16), jnp.float32),
        pltpu.VMEM((NPT,), jnp.float32),
        pltpu.VMEM((NPT,), jnp.float32),
        pltpu.VMEM((NPT,), jnp.float32),
        pltpu.VMEM((2, H), jnp.float32),
    ],
)(_pq_body)


CH = 2000
NCHUNK = N_EDGES // CH


def _edge_body(src_hbm, dst_hbm, ea0_hbm, ea1_hbm, p_hbm, q_hbm, cst_hbm,
               out_hbm, p_v, q_v, sbuf, dbuf, e0, e1, ob, cst_v, semc0, semc1):
    c = lax.axis_index("c")
    s = lax.axis_index("s")
    wid = s * NC + c
    pltpu.sync_copy(p_hbm, p_v)
    pltpu.sync_copy(q_hbm, q_v)
    pltpu.sync_copy(cst_hbm, cst_v)
    w0 = cst_v[0, :]
    w1 = cst_v[1, :]
    bcv = cst_v[2, :]
    sems = (semc0, semc1)
    nmax = (NCHUNK - 1) // NW + 1

    def _base(j):
        return jnp.minimum(wid + j * NW, NCHUNK - 1) * CH

    def _descs(j, b):
        base = _base(j)
        return [
            pltpu.make_async_copy(src_hbm.at[pl.ds(base, CH)], sbuf.at[b],
                                  sems[b]),
            pltpu.make_async_copy(dst_hbm.at[pl.ds(base, CH)], dbuf.at[b],
                                  sems[b]),
            pltpu.make_async_copy(ea0_hbm.at[pl.ds(base, CH)], e0.at[b],
                                  sems[b]),
            pltpu.make_async_copy(ea1_hbm.at[pl.ds(base, CH)], e1.at[b],
                                  sems[b]),
        ]

    def _compute_store(j, b):
        def _grp(g, inner):
            off = g * L
            si = sbuf[b, pl.ds(off, L)]
            di = dbuf[b, pl.ds(off, L)]
            pv = plsc.load_gather(p_v, [si])
            qv = plsc.load_gather(q_v, [di])
            z = (pv + qv + w0 * e0[b, pl.ds(off, L)]
                 + w1 * e1[b, pl.ds(off, L)] + bcv)
            ob[b, pl.ds(off, L)] = 1.0 / (1.0 + jnp.exp(-z))
            return inner

        lax.fori_loop(0, CH // L, _grp, 0)

        @pl.when(wid + j * NW < NCHUNK)
        def _():
            pltpu.sync_copy(ob.at[b], out_hbm.at[pl.ds(_base(j), CH)])

    for d in _descs(0, 0):
        d.start()

    def _pair(i, carry):
        for d in _descs(2 * i + 1, 1):
            d.start()
        for d in _descs(2 * i, 0):
            d.wait()
        _compute_store(2 * i, 0)
        for d in _descs(2 * i + 2, 0):
            d.start()
        for d in _descs(2 * i + 1, 1):
            d.wait()
        _compute_store(2 * i + 1, 1)
        return carry

    lax.fori_loop(0, nmax // 2, _pair, 0)
    for d in _descs(nmax - 1, 0):
        d.wait()
    _compute_store(nmax - 1, 0)


_edges = functools.partial(
    pl.kernel,
    out_type=jax.ShapeDtypeStruct((N_EDGES,), jnp.float32),
    mesh=_mesh(),
    compiler_params=pltpu.CompilerParams(needs_layout_passes=False, use_tc_tiling_on_sc=False),
    scratch_types=[
        pltpu.VMEM((NP,), jnp.float32),
        pltpu.VMEM((NP,), jnp.float32),
        pltpu.VMEM((2, CH), jnp.int32),
        pltpu.VMEM((2, CH), jnp.int32),
        pltpu.VMEM((2, CH), jnp.float32),
        pltpu.VMEM((2, CH), jnp.float32),
        pltpu.VMEM((2, CH), jnp.float32),
        pltpu.VMEM((3, L), jnp.float32),
        pltpu.SemaphoreType.DMA,
        pltpu.SemaphoreType.DMA,
    ],
)(_edge_body)


def kernel(x, edge_index, edge_attr, W1, b1, W2, b2, Wc, bc):
    src = edge_index[0]
    dst = edge_index[1]
    padi = jnp.full((E_PAD - N_EDGES,), NP - 1, jnp.int32)
    src2 = jnp.concatenate([src, padi]).reshape(E_ROWS, EROW)
    dst2 = jnp.concatenate([dst, padi]).reshape(E_ROWS, EROW)
    src4 = jnp.stack([src2, src2 + NP, src2 + 2 * NP, src2 + 3 * NP])
    xcols = [jnp.pad(x[:, k], (0, NP - N_NODES)) for k in range(3)]
    z8 = jnp.zeros((NP, 8), jnp.float32)

    degp = _deg(dst2)
    dinvv, uf = _norm(degp, *xcols)
    u2d = uf.reshape(NP, 8)
    p1 = _agg1(src2, dst2, u2d, z8)
    p1f = p1.reshape(NC, NP * 8)
    g1fl = _dense1(p1f, dinvv, W1.T, b1)
    g1f = g1fl.reshape(4 * NP, 16)
    agg4 = _agg2(src4, dst2, g1f).reshape(4, NP * 16)
    wsd = Wc[0, :2 * H].reshape(2, H)
    m2 = jnp.dot(wsd, W2, precision=_PREC)
    pvec, qvec = _pq(agg4, dinvv, m2)
    bconst = bc[0] + jnp.dot(b2, wsd[0]) + jnp.dot(b2, wsd[1])
    cst = jnp.stack([
        jnp.full((L,), Wc[0, 2 * H], jnp.float32),
        jnp.full((L,), Wc[0, 2 * H + 1], jnp.float32),
        jnp.full((L,), bconst, jnp.float32),
    ])
    out = _edges(src, dst, edge_attr[:, 0], edge_attr[:, 1],
                 pvec, qvec, cst)
    return out[:, None]
